# Initial kernel scaffold; baseline (speedup 1.0000x reference)
import jax
import jax.numpy as jnp
from jax.experimental import pallas as pl


def kernel(idx, flop, tables, W1, b1, W2, b2, Wf, bf):
    raise NotImplementedError("write your pallas kernel here")


# speedup vs baseline: 6.2946x; 6.2946x over previous
import functools

import jax
import jax.numpy as jnp
from jax import lax
from jax.experimental import pallas as pl
from jax.experimental.pallas import tpu as pltpu
from jax.experimental.pallas import tpu_sc as plsc

F = 26
V = 100000
E = 16
FE = F * E
HID = 32
B = 16384
T8P = 12504
T8 = F * T8P

NC, NS = 2, 16
NW = NC * NS
BPW = B // NW
NCH = BPW // 128


NVB = 781
VTAIL = V - NVB * 128


def _sc_transpose(tt, tail):
    mesh = plsc.VectorSubcoreMesh(
        core_axis_name="c", subcore_axis_name="s", num_cores=NC, num_subcores=NS
    )
    nu = F * NVB

    @functools.partial(
        pl.kernel,
        mesh=mesh,
        out_type=jax.ShapeDtypeStruct((T8, 128), jnp.float32),
        scratch_types=[
            pltpu.VMEM((2, 16, 128), jnp.float32),
            pltpu.VMEM((2, 16, 128), jnp.float32),
            pltpu.SemaphoreType.DMA,
        ],
        compiler_params=pltpu.CompilerParams(
            use_tc_tiling_on_sc=True, needs_layout_passes=False
        ),
    )
    def k(tt_hbm, tail_hbm, out_hbm, xb, yb, isem):
        wid = lax.axis_index("c") * NS + lax.axis_index("s")
        iota = lax.iota(jnp.int32, 16)

        def unit(u):
            f = u // NVB
            v0 = (u % NVB) * 128
            return f, v0

        def fetch(u, slot):
            f, v0 = unit(u)
            pltpu.async_copy(
                tt_hbm.at[pl.ds(f * E, E), pl.ds(v0, 128)],
                xb.at[slot],
                isem,
            )

        def wait_fetch(u, slot):
            f, v0 = unit(u)
            pltpu.make_async_copy(
                tt_hbm.at[pl.ds(f * E, E), pl.ds(v0, 128)],
                xb.at[slot],
                isem,
            ).wait()

        nu_w = (nu - wid + NW - 1) // NW

        def body(i, carry):
            u = wid + i * NW
            f, v0 = unit(u)
            slot = i % 2
            slotv = jnp.full((16,), slot, jnp.int32)

            @pl.when(i + 1 < nu_w)
            def _pre():
                fetch(u + NW, (i + 1) % 2)

            wait_fetch(u, slot)

            def col(v, c2):
                vals = plsc.load_gather(
                    xb, [slotv, iota, jnp.full((16,), v, jnp.int32)]
                )
                plsc.store_scatter(
                    yb,
                    [slotv,
                     jnp.full((16,), v // 8, jnp.int32),
                     (v % 8) * 16 + iota],
                    vals,
                )
                return c2

            lax.fori_loop(0, 128, col, 0)
            pltpu.sync_copy(
                yb.at[slot],
                out_hbm.at[pl.ds((f * (T8P // 8) + (u % NVB) * 2) * 8, 16), :],
            )
            return carry

        @pl.when(nu_w > 0)
        def _go():
            fetch(wid, 0)
            lax.fori_loop(0, nu_w, body, 0)

        @pl.when(wid < F)
        def _tail():
            f = wid
            pltpu.sync_copy(tail_hbm.at[pl.ds(f * E, E), :], xb.at[0])
            zv = jnp.zeros((16,), jnp.int32)

            def tcol(v, c2):
                vals = plsc.load_gather(
                    xb, [zv, iota, jnp.full((16,), v, jnp.int32)]
                )
                plsc.store_scatter(
                    yb,
                    [zv,
                     jnp.full((16,), v // 8, jnp.int32),
                     (v % 8) * 16 + iota],
                    vals,
                )
                return c2

            lax.fori_loop(0, VTAIL, tcol, 0)
            pltpu.sync_copy(
                yb.at[0].at[pl.ds(0, VTAIL // 8), :],
                out_hbm.at[
                    pl.ds((f * (T8P // 8) + NVB * 2) * 8, VTAIL // 8), :
                ],
            )

    return k(tt, tail)


def _sc_gather(tables8, idx):
    mesh = plsc.VectorSubcoreMesh(
        core_axis_name="c", subcore_axis_name="s", num_cores=NC, num_subcores=NS
    )

    @functools.partial(
        pl.kernel,
        mesh=mesh,
        out_type=jax.ShapeDtypeStruct((B, FE), jnp.float32),
        scratch_types=[pltpu.VMEM((BPW,), jnp.int32)]
        + [pltpu.VMEM((128,), jnp.int32) for _ in range(NCH)]
        + [pltpu.VMEM((128, 128), jnp.float32) for _ in range(NCH)]
        + [pltpu.VMEM((BPW, E), jnp.float32), pltpu.SemaphoreType.DMA],
        compiler_params=pltpu.CompilerParams(
            use_tc_tiling_on_sc=False, needs_layout_passes=False
        ),
    )
    def k(tab_hbm, idx_hbm, feats_hbm, idx_v, a0, a1, a2, a3,
          r0, r1, r2, r3, rows_v, gsem):
        wid = lax.axis_index("c") * NS + lax.axis_index("s")
        base = wid * BPW
        aidx = [a0, a1, a2, a3]
        rbuf = [r0, r1, r2, r3]
        iota = lax.iota(jnp.int32, 16)

        def body(f, carry):
            pltpu.sync_copy(idx_hbm.at[f, pl.ds(base, BPW)], idx_v)
            for c in range(NCH):
                for h in range(8):
                    iv = idx_v[pl.ds(c * 128 + h * 16, 16)]
                    aidx[c][pl.ds(h * 16, 16)] = jax.lax.shift_right_logical(
                        iv, 3
                    ) + f * T8P
            cps = [
                pltpu.async_copy(
                    tab_hbm.at[aidx[c]],
                    rbuf[c],
                    gsem,
                )
                for c in range(NCH)
            ]
            for cp in cps:
                cp.wait()
            for c in range(NCH):
                for h in range(8):
                    iv = idx_v[pl.ds(c * 128 + h * 16, 16)]
                    rows16 = h * 16 + iota
                    colbase = (iv & 7) * 16
                    maskf = (iv != 0).astype(jnp.float32)
                    for j in range(E):
                        vals = plsc.load_gather(
                            rbuf[c], [rows16, colbase + j]
                        )
                        plsc.store_scatter(
                            rows_v,
                            [c * 128 + rows16, jnp.full((16,), j, jnp.int32)],
                            vals * maskf,
                        )

            pltpu.sync_copy(
                rows_v, feats_hbm.at[pl.ds(base, BPW), pl.ds(f * E, E)]
            )
            return carry

        lax.fori_loop(0, F, body, 0)

    return k(tables8, idx)


def _mlp(feats, flop2, W1a, w1b, b1r, W2, b2r, Wf, bfr):
    BB = 2048

    def body(f_ref, fl_ref, w1a_ref, w1b_ref, b1_ref, w2_ref, b2_ref, wf_ref,
             bf_ref, o_ref):
        x2 = jnp.clip(fl_ref[...] * w2_ref[...] + b2_ref[...], 0.0, 6.0)
        pre = jnp.dot(f_ref[...], w1a_ref[...],
                      preferred_element_type=jnp.float32)
        pre = pre + x2 * w1b_ref[...] + b1_ref[...]
        h = jnp.clip(pre, 0.0, 6.0)
        o_ref[...] = (
            jnp.dot(h, wf_ref[...], preferred_element_type=jnp.float32)
            + bf_ref[...]
        )

    return pl.pallas_call(
        body,
        grid=(B // BB,),
        in_specs=[
            pl.BlockSpec((BB, FE), lambda i: (i, 0)),
            pl.BlockSpec((BB, 1), lambda i: (i, 0)),
            pl.BlockSpec((FE, HID), lambda i: (0, 0)),
            pl.BlockSpec((1, HID), lambda i: (0, 0)),
            pl.BlockSpec((1, HID), lambda i: (0, 0)),
            pl.BlockSpec((1, 1), lambda i: (0, 0)),
            pl.BlockSpec((1, 1), lambda i: (0, 0)),
            pl.BlockSpec((HID, 1), lambda i: (0, 0)),
            pl.BlockSpec((1, 1), lambda i: (0, 0)),
        ],
        out_specs=pl.BlockSpec((BB, 1), lambda i: (i, 0)),
        out_shape=jax.ShapeDtypeStruct((B, 1), jnp.float32),
    )(feats, flop2, W1a, w1b, b1r, W2, b2r, Wf, bfr)


def kernel(idx, flop, tables, W1, b1, W2, b2, Wf, bf):
    idx = idx.astype(jnp.int32)
    tt = tables.transpose(0, 2, 1).reshape(FE, V)
    tail = jnp.pad(tt[:, NVB * 128:], ((0, 0), (0, 128 - VTAIL)))
    tab8 = _sc_transpose(tt, tail)
    feats = _sc_gather(tab8, idx)
    flop2 = flop.reshape(B, 1).astype(jnp.float32)
    W1a = W1[:FE]
    w1b = W1[FE:].reshape(1, HID)
    return _mlp(
        feats,
        flop2,
        W1a,
        w1b,
        b1.reshape(1, HID),
        W2,
        b2.reshape(1, 1),
        Wf,
        bf.reshape(1, 1),
    )


# speedup vs baseline: 6.3069x; 1.0020x over previous
import functools

import jax
import jax.numpy as jnp
from jax import lax
from jax.experimental import pallas as pl
from jax.experimental.pallas import tpu as pltpu
from jax.experimental.pallas import tpu_sc as plsc

F = 26
V = 100000
E = 16
FE = F * E
HID = 32
B = 16384
T8P = 12504
T8 = F * T8P

NC, NS = 2, 16
NW = NC * NS
BPW = B // NW
NCH = BPW // 128


NVB = 781
VTAIL = V - NVB * 128


def _sc_transpose(tt, tail):
    mesh = plsc.VectorSubcoreMesh(
        core_axis_name="c", subcore_axis_name="s", num_cores=NC, num_subcores=NS
    )
    nu = F * NVB

    @functools.partial(
        pl.kernel,
        mesh=mesh,
        out_type=jax.ShapeDtypeStruct((T8, 128), jnp.float32),
        scratch_types=[
            pltpu.VMEM((2, 16, 128), jnp.float32),
            pltpu.VMEM((2, 16, 128), jnp.float32),
            pltpu.SemaphoreType.DMA,
        ],
        compiler_params=pltpu.CompilerParams(
            use_tc_tiling_on_sc=True, needs_layout_passes=False
        ),
    )
    def k(tt_hbm, tail_hbm, out_hbm, xb, yb, isem):
        wid = lax.axis_index("c") * NS + lax.axis_index("s")
        iota = lax.iota(jnp.int32, 16)

        def unit(u):
            f = u // NVB
            v0 = (u % NVB) * 128
            return f, v0

        def fetch(u, slot):
            f, v0 = unit(u)
            pltpu.async_copy(
                tt_hbm.at[pl.ds(f * E, E), pl.ds(v0, 128)],
                xb.at[slot],
                isem,
            )

        def wait_fetch(u, slot):
            f, v0 = unit(u)
            pltpu.make_async_copy(
                tt_hbm.at[pl.ds(f * E, E), pl.ds(v0, 128)],
                xb.at[slot],
                isem,
            ).wait()

        nu_w = (nu - wid + NW - 1) // NW

        def body(i, carry):
            u = wid + i * NW
            f, v0 = unit(u)
            slot = i % 2
            slotv = jnp.full((16,), slot, jnp.int32)

            @pl.when(i + 1 < nu_w)
            def _pre():
                fetch(u + NW, (i + 1) % 2)

            wait_fetch(u, slot)

            def col(v, c2):
                vals = plsc.load_gather(
                    xb, [slotv, iota, jnp.full((16,), v, jnp.int32)]
                )
                plsc.store_scatter(
                    yb,
                    [slotv,
                     jnp.full((16,), v // 8, jnp.int32),
                     (v % 8) * 16 + iota],
                    vals,
                )
                return c2

            lax.fori_loop(0, 128, col, 0, unroll=8)
            pltpu.sync_copy(
                yb.at[slot],
                out_hbm.at[pl.ds((f * (T8P // 8) + (u % NVB) * 2) * 8, 16), :],
            )
            return carry

        @pl.when(nu_w > 0)
        def _go():
            fetch(wid, 0)
            lax.fori_loop(0, nu_w, body, 0)

        @pl.when(wid < F)
        def _tail():
            f = wid
            pltpu.sync_copy(tail_hbm.at[pl.ds(f * E, E), :], xb.at[0])
            zv = jnp.zeros((16,), jnp.int32)

            def tcol(v, c2):
                vals = plsc.load_gather(
                    xb, [zv, iota, jnp.full((16,), v, jnp.int32)]
                )
                plsc.store_scatter(
                    yb,
                    [zv,
                     jnp.full((16,), v // 8, jnp.int32),
                     (v % 8) * 16 + iota],
                    vals,
                )
                return c2

            lax.fori_loop(0, VTAIL, tcol, 0)
            pltpu.sync_copy(
                yb.at[0].at[pl.ds(0, VTAIL // 8), :],
                out_hbm.at[
                    pl.ds((f * (T8P // 8) + NVB * 2) * 8, VTAIL // 8), :
                ],
            )

    return k(tt, tail)


def _sc_gather(tables8, idx):
    mesh = plsc.VectorSubcoreMesh(
        core_axis_name="c", subcore_axis_name="s", num_cores=NC, num_subcores=NS
    )

    @functools.partial(
        pl.kernel,
        mesh=mesh,
        out_type=jax.ShapeDtypeStruct((B, FE), jnp.float32),
        scratch_types=[pltpu.VMEM((BPW,), jnp.int32)]
        + [pltpu.VMEM((128,), jnp.int32) for _ in range(NCH)]
        + [pltpu.VMEM((128, 128), jnp.float32) for _ in range(NCH)]
        + [pltpu.VMEM((BPW, E), jnp.float32), pltpu.SemaphoreType.DMA],
        compiler_params=pltpu.CompilerParams(
            use_tc_tiling_on_sc=False, needs_layout_passes=False
        ),
    )
    def k(tab_hbm, idx_hbm, feats_hbm, idx_v, a0, a1, a2, a3,
          r0, r1, r2, r3, rows_v, gsem):
        wid = lax.axis_index("c") * NS + lax.axis_index("s")
        base = wid * BPW
        aidx = [a0, a1, a2, a3]
        rbuf = [r0, r1, r2, r3]
        iota = lax.iota(jnp.int32, 16)

        def body(f, carry):
            pltpu.sync_copy(idx_hbm.at[f, pl.ds(base, BPW)], idx_v)
            for c in range(NCH):
                for h in range(8):
                    iv = idx_v[pl.ds(c * 128 + h * 16, 16)]
                    aidx[c][pl.ds(h * 16, 16)] = jax.lax.shift_right_logical(
                        iv, 3
                    ) + f * T8P
            cps = [
                pltpu.async_copy(
                    tab_hbm.at[aidx[c]],
                    rbuf[c],
                    gsem,
                )
                for c in range(NCH)
            ]
            for cp in cps:
                cp.wait()
            for c in range(NCH):
                for h in range(8):
                    iv = idx_v[pl.ds(c * 128 + h * 16, 16)]
                    rows16 = h * 16 + iota
                    colbase = (iv & 7) * 16
                    maskf = (iv != 0).astype(jnp.float32)
                    for j in range(E):
                        vals = plsc.load_gather(
                            rbuf[c], [rows16, colbase + j]
                        )
                        plsc.store_scatter(
                            rows_v,
                            [c * 128 + rows16, jnp.full((16,), j, jnp.int32)],
                            vals * maskf,
                        )

            pltpu.sync_copy(
                rows_v, feats_hbm.at[pl.ds(base, BPW), pl.ds(f * E, E)]
            )
            return carry

        lax.fori_loop(0, F, body, 0)

    return k(tables8, idx)


def _mlp(feats, flop2, W1a, w1b, b1r, W2, b2r, Wf, bfr):
    BB = 2048

    def body(f_ref, fl_ref, w1a_ref, w1b_ref, b1_ref, w2_ref, b2_ref, wf_ref,
             bf_ref, o_ref):
        x2 = jnp.clip(fl_ref[...] * w2_ref[...] + b2_ref[...], 0.0, 6.0)
        pre = jnp.dot(f_ref[...], w1a_ref[...],
                      preferred_element_type=jnp.float32)
        pre = pre + x2 * w1b_ref[...] + b1_ref[...]
        h = jnp.clip(pre, 0.0, 6.0)
        o_ref[...] = (
            jnp.dot(h, wf_ref[...], preferred_element_type=jnp.float32)
            + bf_ref[...]
        )

    return pl.pallas_call(
        body,
        grid=(B // BB,),
        in_specs=[
            pl.BlockSpec((BB, FE), lambda i: (i, 0)),
            pl.BlockSpec((BB, 1), lambda i: (i, 0)),
            pl.BlockSpec((FE, HID), lambda i: (0, 0)),
            pl.BlockSpec((1, HID), lambda i: (0, 0)),
            pl.BlockSpec((1, HID), lambda i: (0, 0)),
            pl.BlockSpec((1, 1), lambda i: (0, 0)),
            pl.BlockSpec((1, 1), lambda i: (0, 0)),
            pl.BlockSpec((HID, 1), lambda i: (0, 0)),
            pl.BlockSpec((1, 1), lambda i: (0, 0)),
        ],
        out_specs=pl.BlockSpec((BB, 1), lambda i: (i, 0)),
        out_shape=jax.ShapeDtypeStruct((B, 1), jnp.float32),
    )(feats, flop2, W1a, w1b, b1r, W2, b2r, Wf, bfr)


def kernel(idx, flop, tables, W1, b1, W2, b2, Wf, bf):
    idx = idx.astype(jnp.int32)
    tt = tables.transpose(0, 2, 1).reshape(FE, V)
    tail = jnp.pad(tt[:, NVB * 128:], ((0, 0), (0, 128 - VTAIL)))
    tab8 = _sc_transpose(tt, tail)
    feats = _sc_gather(tab8, idx)
    flop2 = flop.reshape(B, 1).astype(jnp.float32)
    W1a = W1[:FE]
    w1b = W1[FE:].reshape(1, HID)
    return _mlp(
        feats,
        flop2,
        W1a,
        w1b,
        b1.reshape(1, HID),
        W2,
        b2.reshape(1, 1),
        Wf,
        bf.reshape(1, 1),
    )


# speedup vs baseline: 6.5446x; 1.0377x over previous
import functools

import jax
import jax.numpy as jnp
from jax import lax
from jax.experimental import pallas as pl
from jax.experimental.pallas import tpu as pltpu
from jax.experimental.pallas import tpu_sc as plsc

F = 26
V = 100000
E = 16
FE = F * E
HID = 32
B = 16384
T8P = 12504
T8 = F * T8P

NC, NS = 2, 16
NW = NC * NS
BPW = B // NW
NCH = BPW // 128


NVB = 781
VTAIL = V - NVB * 128


def _sc_transpose(tt, tail):
    mesh = plsc.VectorSubcoreMesh(
        core_axis_name="c", subcore_axis_name="s", num_cores=NC, num_subcores=NS
    )
    nu = F * NVB

    @functools.partial(
        pl.kernel,
        mesh=mesh,
        out_type=jax.ShapeDtypeStruct((T8, 128), jnp.float32),
        scratch_types=[
            pltpu.VMEM((2, 16, 128), jnp.float32),
            pltpu.VMEM((2, 16, 128), jnp.float32),
            pltpu.SemaphoreType.DMA,
        ],
        compiler_params=pltpu.CompilerParams(
            use_tc_tiling_on_sc=True, needs_layout_passes=False
        ),
    )
    def k(tt_hbm, tail_hbm, out_hbm, xb, yb, isem):
        wid = lax.axis_index("c") * NS + lax.axis_index("s")
        iota = lax.iota(jnp.int32, 16)

        def unit(u):
            f = u // NVB
            v0 = (u % NVB) * 128
            return f, v0

        def fetch(u, slot):
            f, v0 = unit(u)
            pltpu.async_copy(
                tt_hbm.at[pl.ds(f * E, E), pl.ds(v0, 128)],
                xb.at[slot],
                isem,
            )

        def wait_fetch(u, slot):
            f, v0 = unit(u)
            pltpu.make_async_copy(
                tt_hbm.at[pl.ds(f * E, E), pl.ds(v0, 128)],
                xb.at[slot],
                isem,
            ).wait()

        nu_w = (nu - wid + NW - 1) // NW

        def body(i, carry):
            u = wid + i * NW
            f, v0 = unit(u)
            slot = i % 2
            slotv = jnp.full((16,), slot, jnp.int32)

            @pl.when(i + 1 < nu_w)
            def _pre():
                fetch(u + NW, (i + 1) % 2)

            wait_fetch(u, slot)

            def col(v, c2):
                vals = plsc.load_gather(
                    xb, [slotv, iota, jnp.full((16,), v, jnp.int32)]
                )
                plsc.store_scatter(
                    yb,
                    [slotv,
                     jnp.full((16,), v // 8, jnp.int32),
                     (v % 8) * 16 + iota],
                    vals,
                )
                return c2

            lax.fori_loop(0, 128, col, 0, unroll=8)
            pltpu.sync_copy(
                yb.at[slot],
                out_hbm.at[pl.ds((f * (T8P // 8) + (u % NVB) * 2) * 8, 16), :],
            )
            return carry

        @pl.when(nu_w > 0)
        def _go():
            fetch(wid, 0)
            lax.fori_loop(0, nu_w, body, 0)

        @pl.when(wid < F)
        def _tail():
            f = wid
            pltpu.sync_copy(tail_hbm.at[pl.ds(f * E, E), :], xb.at[0])
            zv = jnp.zeros((16,), jnp.int32)

            def tcol(v, c2):
                vals = plsc.load_gather(
                    xb, [zv, iota, jnp.full((16,), v, jnp.int32)]
                )
                plsc.store_scatter(
                    yb,
                    [zv,
                     jnp.full((16,), v // 8, jnp.int32),
                     (v % 8) * 16 + iota],
                    vals,
                )
                return c2

            lax.fori_loop(0, VTAIL, tcol, 0)
            pltpu.sync_copy(
                yb.at[0].at[pl.ds(0, VTAIL // 8), :],
                out_hbm.at[
                    pl.ds((f * (T8P // 8) + NVB * 2) * 8, VTAIL // 8), :
                ],
            )

    return k(tt, tail)


def _sc_gather(tables8, idx):
    mesh = plsc.VectorSubcoreMesh(
        core_axis_name="c", subcore_axis_name="s", num_cores=NC, num_subcores=NS
    )

    @functools.partial(
        pl.kernel,
        mesh=mesh,
        out_type=jax.ShapeDtypeStruct((B, FE), jnp.float32),
        scratch_types=[pltpu.VMEM((BPW,), jnp.int32)]
        + [pltpu.VMEM((128,), jnp.int32) for _ in range(NCH)]
        + [pltpu.VMEM((128, 128), jnp.float32) for _ in range(NCH)]
        + [pltpu.VMEM((BPW, E), jnp.float32), pltpu.SemaphoreType.DMA],
        compiler_params=pltpu.CompilerParams(
            use_tc_tiling_on_sc=False, needs_layout_passes=False
        ),
    )
    def k(tab_hbm, idx_hbm, feats_hbm, idx_v, a0, a1, a2, a3,
          r0, r1, r2, r3, rows_v, gsem):
        wid = lax.axis_index("c") * NS + lax.axis_index("s")
        base = wid * BPW
        aidx = [a0, a1, a2, a3]
        rbuf = [r0, r1, r2, r3]
        iota = lax.iota(jnp.int32, 16)

        def body(f, carry):
            pltpu.sync_copy(idx_hbm.at[f, pl.ds(base, BPW)], idx_v)
            for c in range(NCH):
                for h in range(8):
                    iv = idx_v[pl.ds(c * 128 + h * 16, 16)]
                    aidx[c][pl.ds(h * 16, 16)] = jax.lax.shift_right_logical(
                        iv, 3
                    ) + f * (V // 8)
            cps = [
                pltpu.async_copy(
                    tab_hbm.at[aidx[c]],
                    rbuf[c],
                    gsem,
                )
                for c in range(NCH)
            ]
            for cp in cps:
                cp.wait()
            for c in range(NCH):
                for h in range(8):
                    iv = idx_v[pl.ds(c * 128 + h * 16, 16)]
                    rows16 = h * 16 + iota
                    colbase = (iv & 7) * 16
                    maskf = (iv != 0).astype(jnp.float32)
                    for j in range(E):
                        vals = plsc.load_gather(
                            rbuf[c], [rows16, colbase + j]
                        )
                        plsc.store_scatter(
                            rows_v,
                            [c * 128 + rows16, jnp.full((16,), j, jnp.int32)],
                            vals * maskf,
                        )

            pltpu.sync_copy(
                rows_v, feats_hbm.at[pl.ds(base, BPW), pl.ds(f * E, E)]
            )
            return carry

        lax.fori_loop(0, F, body, 0)

    return k(tables8, idx)


def _mlp(feats, flop2, W1a, w1b, b1r, W2, b2r, Wf, bfr):
    BB = 2048

    def body(f_ref, fl_ref, w1a_ref, w1b_ref, b1_ref, w2_ref, b2_ref, wf_ref,
             bf_ref, o_ref):
        x2 = jnp.clip(fl_ref[...] * w2_ref[...] + b2_ref[...], 0.0, 6.0)
        pre = jnp.dot(f_ref[...], w1a_ref[...],
                      preferred_element_type=jnp.float32)
        pre = pre + x2 * w1b_ref[...] + b1_ref[...]
        h = jnp.clip(pre, 0.0, 6.0)
        o_ref[...] = (
            jnp.dot(h, wf_ref[...], preferred_element_type=jnp.float32)
            + bf_ref[...]
        )

    return pl.pallas_call(
        body,
        grid=(B // BB,),
        in_specs=[
            pl.BlockSpec((BB, FE), lambda i: (i, 0)),
            pl.BlockSpec((BB, 1), lambda i: (i, 0)),
            pl.BlockSpec((FE, HID), lambda i: (0, 0)),
            pl.BlockSpec((1, HID), lambda i: (0, 0)),
            pl.BlockSpec((1, HID), lambda i: (0, 0)),
            pl.BlockSpec((1, 1), lambda i: (0, 0)),
            pl.BlockSpec((1, 1), lambda i: (0, 0)),
            pl.BlockSpec((HID, 1), lambda i: (0, 0)),
            pl.BlockSpec((1, 1), lambda i: (0, 0)),
        ],
        out_specs=pl.BlockSpec((BB, 1), lambda i: (i, 0)),
        out_shape=jax.ShapeDtypeStruct((B, 1), jnp.float32),
    )(feats, flop2, W1a, w1b, b1r, W2, b2r, Wf, bfr)


def kernel(idx, flop, tables, W1, b1, W2, b2, Wf, bf):
    idx = idx.astype(jnp.int32)
    tab8 = tables.reshape(F * V // 8, 8 * E)
    feats = _sc_gather(tab8, idx)
    flop2 = flop.reshape(B, 1).astype(jnp.float32)
    W1a = W1[:FE]
    w1b = W1[FE:].reshape(1, HID)
    return _mlp(
        feats,
        flop2,
        W1a,
        w1b,
        b1.reshape(1, HID),
        W2,
        b2.reshape(1, 1),
        Wf,
        bf.reshape(1, 1),
    )
